# Initial kernel scaffold; baseline (speedup 1.0000x reference)
import jax
import jax.numpy as jnp
from jax.experimental import pallas as pl


def kernel(feature, W_conv, b_conv, W_cls, b_cls, W_reg, b_reg):
    raise NotImplementedError("write your pallas kernel here")


# speedup vs baseline: 1.2766x; 1.2766x over previous
import jax
import jax.numpy as jnp
from jax.experimental import pallas as pl

_H = 100
_W = 100
_WP = _W + 2
_NVALID = _H * _WP
_TILE = 1024
_NCOL = 10240
_NTILE = _NCOL // _TILE
_NPAD = 10496
_OFFS = tuple(ky * _WP + kx for ky in range(3) for kx in range(3))
_HEAD_PAD = 48


def _rpn_body(x_ref, wt_ref, wh_ref, bc_ref, bh_ref, out_ref):
    x = x_ref[0]
    bc = bc_ref[:, :]
    bh = bh_ref[:, :]
    wh = wh_ref[:, :]
    for t in range(_NTILE):
        base = t * _TILE
        acc = jnp.dot(wt_ref[0], x[:, base + _OFFS[0]: base + _OFFS[0] + _TILE],
                      preferred_element_type=jnp.float32)
        for k in range(1, 9):
            off = _OFFS[k]
            acc = acc + jnp.dot(wt_ref[k], x[:, base + off: base + off + _TILE],
                                preferred_element_type=jnp.float32)
        h = jnp.maximum(acc + bc, 0.0).astype(jnp.bfloat16)
        o = jnp.dot(wh, h, preferred_element_type=jnp.float32) + bh
        out_ref[0, :, base: base + _TILE] = o


def kernel(feature, W_conv, b_conv, W_cls, b_cls, W_reg, b_reg):
    B, C, H, W = feature.shape
    A = W_cls.shape[0]
    R = W_reg.shape[0]

    xp = jnp.pad(feature, ((0, 0), (0, 0), (1, 1), (1, 1)))
    xp = xp.reshape(B, C, (H + 2) * (W + 2))
    xp = jnp.pad(xp, ((0, 0), (0, 0), (0, _NPAD - (H + 2) * (W + 2))))
    xp = xp.astype(jnp.bfloat16)

    wt = W_conv.transpose(2, 3, 0, 1).reshape(9, C, C).astype(jnp.bfloat16)

    whead = jnp.concatenate([W_cls.reshape(A, C), W_reg.reshape(R, C)], axis=0)
    whead = jnp.pad(whead, ((0, _HEAD_PAD - (A + R)), (0, 0))).astype(jnp.bfloat16)
    bhead = jnp.concatenate([b_cls, b_reg])
    bhead = jnp.pad(bhead, (0, _HEAD_PAD - (A + R))).reshape(_HEAD_PAD, 1)
    bconv = b_conv.reshape(C, 1)

    out = pl.pallas_call(
        _rpn_body,
        grid=(B,),
        in_specs=[
            pl.BlockSpec((1, C, _NPAD), lambda b: (b, 0, 0)),
            pl.BlockSpec((9, C, C), lambda b: (0, 0, 0)),
            pl.BlockSpec((_HEAD_PAD, C), lambda b: (0, 0)),
            pl.BlockSpec((C, 1), lambda b: (0, 0)),
            pl.BlockSpec((_HEAD_PAD, 1), lambda b: (0, 0)),
        ],
        out_specs=pl.BlockSpec((1, _HEAD_PAD, _NCOL), lambda b: (b, 0, 0)),
        out_shape=jax.ShapeDtypeStruct((B, _HEAD_PAD, _NCOL), jnp.float32),
    )(xp, wt, whead, bconv, bhead)

    res = out[:, :A + R, :_NVALID].reshape(B, A + R, H, _WP)[:, :, :, :W]
    return res


# speedup vs baseline: 1.3343x; 1.0451x over previous
import jax
import jax.numpy as jnp
from jax.experimental import pallas as pl
from jax.experimental.pallas import tpu as pltpu

_H = 100
_W = 100
_WP = _W + 2
_ROWS_PER_TILE = 10
_TILE = _ROWS_PER_TILE * _WP
_NTILE = _H // _ROWS_PER_TILE
_NPAD = 10496
_OFFS = tuple(ky * _WP + kx for ky in range(3) for kx in range(3))
_NHEAD = 45
_HEAD_PAD = 48


def _rpn_body(x_ref, wt_ref, wh_ref, bc_ref, bh_ref, out_ref, xp_ref):
    C = x_ref.shape[1]
    x = x_ref[0]
    bc = bc_ref[:, :]
    bh = bh_ref[:, :]
    wh = wh_ref[:, :]

    xp_ref[...] = jnp.zeros((C, _NPAD), jnp.bfloat16)
    for h in range(_H):
        xp_ref[:, h * _WP + _WP + 1: h * _WP + _WP + 1 + _W] = (
            x[:, h * _W: h * _W + _W].astype(jnp.bfloat16))

    for t in range(_NTILE):
        base = t * _TILE
        acc = jnp.dot(wt_ref[0], xp_ref[:, base: base + _TILE],
                      preferred_element_type=jnp.float32)
        for k in range(1, 9):
            off = _OFFS[k]
            acc = acc + jnp.dot(wt_ref[k], xp_ref[:, base + off: base + off + _TILE],
                                preferred_element_type=jnp.float32)
        hid = jnp.maximum(acc + bc, 0.0).astype(jnp.bfloat16)
        o = jnp.dot(wh, hid, preferred_element_type=jnp.float32) + bh
        for r in range(_ROWS_PER_TILE):
            out_ref[0, :, t * _ROWS_PER_TILE * _W + r * _W:
                    t * _ROWS_PER_TILE * _W + r * _W + _W] = (
                o[:_NHEAD, r * _WP: r * _WP + _W])


def kernel(feature, W_conv, b_conv, W_cls, b_cls, W_reg, b_reg):
    B, C, H, W = feature.shape
    A = W_cls.shape[0]
    R = W_reg.shape[0]

    xf = feature.reshape(B, C, H * W)

    wt = W_conv.transpose(2, 3, 0, 1).reshape(9, C, C).astype(jnp.bfloat16)

    whead = jnp.concatenate([W_cls.reshape(A, C), W_reg.reshape(R, C)], axis=0)
    whead = jnp.pad(whead, ((0, _HEAD_PAD - (A + R)), (0, 0))).astype(jnp.bfloat16)
    bhead = jnp.concatenate([b_cls, b_reg])
    bhead = jnp.pad(bhead, (0, _HEAD_PAD - (A + R))).reshape(_HEAD_PAD, 1)
    bconv = b_conv.reshape(C, 1)

    out = pl.pallas_call(
        _rpn_body,
        grid=(B,),
        in_specs=[
            pl.BlockSpec((1, C, H * W), lambda b: (b, 0, 0)),
            pl.BlockSpec((9, C, C), lambda b: (0, 0, 0)),
            pl.BlockSpec((_HEAD_PAD, C), lambda b: (0, 0)),
            pl.BlockSpec((C, 1), lambda b: (0, 0)),
            pl.BlockSpec((_HEAD_PAD, 1), lambda b: (0, 0)),
        ],
        out_specs=pl.BlockSpec((1, _NHEAD, H * W), lambda b: (b, 0, 0)),
        out_shape=jax.ShapeDtypeStruct((B, _NHEAD, H * W), jnp.float32),
        scratch_shapes=[pltpu.VMEM((C, _NPAD), jnp.bfloat16)],
    )(xf, wt, whead, bconv, bhead)

    return out.reshape(B, A + R, H, W)


# speedup vs baseline: 1.3884x; 1.0406x over previous
import jax
import jax.numpy as jnp
from jax.experimental import pallas as pl
from jax.experimental.pallas import tpu as pltpu

_H = 100
_W = 100
_WP = _W + 2
_ROWS_PER_TILE = 10
_TILE = _ROWS_PER_TILE * _WP
_NTILE = _H // _ROWS_PER_TILE
_NPAD = 10496
_OFFS = tuple(ky * _WP + kx for ky in range(3) for kx in range(3))
_NHEAD = 45
_HEAD_PAD = 48


def _rpn_body(x_ref, wt_ref, wh_ref, bc_ref, bh_ref, out_ref, xp_ref):
    C = x_ref.shape[2]
    bc = bc_ref[:, :]
    bh = bh_ref[:, :]
    wh = wh_ref[:, :]

    xp_ref[...] = jnp.zeros((C, _NPAD), jnp.bfloat16)
    for h in range(_H):
        xp_ref[:, h * _WP + _WP + 1: h * _WP + _WP + 1 + _W] = x_ref[0, h]

    for t in range(_NTILE):
        base = t * _TILE
        acc = jnp.dot(wt_ref[0], xp_ref[:, base: base + _TILE],
                      preferred_element_type=jnp.float32)
        for k in range(1, 9):
            off = _OFFS[k]
            acc = acc + jnp.dot(wt_ref[k], xp_ref[:, base + off: base + off + _TILE],
                                preferred_element_type=jnp.float32)
        hid = jnp.maximum(acc + bc, 0.0).astype(jnp.bfloat16)
        o = jnp.dot(wh, hid, preferred_element_type=jnp.float32) + bh
        for r in range(_ROWS_PER_TILE):
            out_ref[0, :, t * _ROWS_PER_TILE * _W + r * _W:
                    t * _ROWS_PER_TILE * _W + r * _W + _W] = (
                o[:_NHEAD, r * _WP: r * _WP + _W])


def kernel(feature, W_conv, b_conv, W_cls, b_cls, W_reg, b_reg):
    B, C, H, W = feature.shape
    A = W_cls.shape[0]
    R = W_reg.shape[0]

    xt = feature.transpose(0, 2, 1, 3).astype(jnp.bfloat16)

    wt = W_conv.transpose(2, 3, 0, 1).reshape(9, C, C).astype(jnp.bfloat16)

    whead = jnp.concatenate([W_cls.reshape(A, C), W_reg.reshape(R, C)], axis=0)
    whead = jnp.pad(whead, ((0, _HEAD_PAD - (A + R)), (0, 0))).astype(jnp.bfloat16)
    bhead = jnp.concatenate([b_cls, b_reg])
    bhead = jnp.pad(bhead, (0, _HEAD_PAD - (A + R))).reshape(_HEAD_PAD, 1)
    bconv = b_conv.reshape(C, 1)

    out = pl.pallas_call(
        _rpn_body,
        grid=(B,),
        in_specs=[
            pl.BlockSpec((1, H, C, W), lambda b: (b, 0, 0, 0)),
            pl.BlockSpec((9, C, C), lambda b: (0, 0, 0)),
            pl.BlockSpec((_HEAD_PAD, C), lambda b: (0, 0)),
            pl.BlockSpec((C, 1), lambda b: (0, 0)),
            pl.BlockSpec((_HEAD_PAD, 1), lambda b: (0, 0)),
        ],
        out_specs=pl.BlockSpec((1, _NHEAD, H * W), lambda b: (b, 0, 0)),
        out_shape=jax.ShapeDtypeStruct((B, _NHEAD, H * W), jnp.float32),
        scratch_shapes=[pltpu.VMEM((C, _NPAD), jnp.bfloat16)],
    )(xt, wt, whead, bconv, bhead)

    return out.reshape(B, A + R, H, W)
